# baseline (speedup 1.0000x reference)
import math
from functools import partial

import jax
import jax.numpy as jnp
from jax.experimental import pallas as pl
from jax.experimental.pallas import tpu as pltpu


def _layer1_cast_kernel(a_ref, x_ref, w1_ref, b1_ref, h1_ref, abf_ref):
    a = a_ref[...]
    abf_ref[...] = a.astype(jnp.bfloat16)
    x = x_ref[...]
    acc = b1_ref[...]
    for r in range(a_ref.shape[0]):
        z = jnp.dot(a[r], x, preferred_element_type=jnp.float32)
        acc = acc + jnp.dot(z, w1_ref[r], preferred_element_type=jnp.float32)
    h1_ref[...] = jnp.maximum(acc, 0.0).astype(h1_ref.dtype)


def _layer2_kernel(a_ref, h1_ref, w2_ref, b2_ref, pool_ref, wc_ref, bc_ref,
                   out_ref, hg_s):
    i = pl.program_id(0)
    h1 = h1_ref[...]
    acc = b2_ref[...]
    for r in range(a_ref.shape[0]):
        z = jnp.dot(a_ref[r], h1, preferred_element_type=jnp.float32)
        acc = acc + jnp.dot(z, w2_ref[r], preferred_element_type=jnp.float32)
    part = jnp.dot(pool_ref[...], acc, preferred_element_type=jnp.float32)

    @pl.when(i == 0)
    def _init():
        hg_s[...] = part

    @pl.when(i > 0)
    def _acc():
        hg_s[...] += part

    @pl.when(i == pl.num_programs(0) - 1)
    def _classifier():
        out_ref[...] = (jnp.dot(hg_s[...], wc_ref[...],
                                preferred_element_type=jnp.float32)
                        + bc_ref[0:1])


def _pad_to(a, shape):
    return jnp.pad(a, [(0, t - s) for s, t in zip(a.shape, shape)])


@partial(jax.jit, static_argnames=("tile_m",))
def _forward(a_norm, x, w1, b1_node, w2, b2_node, pool, wc, bc, *, tile_m=256):
    n_rel, n, _ = a_norm.shape
    f_in = x.shape[1]
    f_hid = w1.shape[2]
    n_graphs = pool.shape[0]
    n_classes = wc.shape[1]

    n_pad = tile_m * pl.cdiv(n, tile_m)
    g_pad = 8 * pl.cdiv(n_graphs, 8)
    n_tiles = n_pad // tile_m

    a_p = _pad_to(a_norm.astype(jnp.float32), (n_rel, n_pad, n_pad))
    x_p = _pad_to(x.astype(jnp.float32), (n_pad, f_in))
    w1_p = jnp.asarray(w1, jnp.float32)
    w2_p = jnp.asarray(w2, jnp.float32)
    b1_p = _pad_to(b1_node.astype(jnp.float32), (n_pad, f_hid))
    b2_p = _pad_to(b2_node.astype(jnp.float32), (n_pad, f_hid))
    pool_p = _pad_to(pool.astype(jnp.float32), (g_pad, n_pad))
    wc_p = jnp.asarray(wc, jnp.float32)
    bc_p = jnp.tile(jnp.asarray(bc, jnp.float32)[None, :], (8, 1))

    slab_f32 = n_rel * tile_m * n_pad * 4
    resident = (n_pad * (f_in + f_hid) * 4 + n_rel * f_hid * f_hid * 8
                + 4 * tile_m * f_hid * 4 + 4 * g_pad * (tile_m + f_hid) * 4)
    vmem1 = int(min(3 * slab_f32 + resident + (8 << 20), 62 << 20))
    vmem2 = int(min(slab_f32 + resident + (8 << 20), 62 << 20))
    cp1 = pltpu.CompilerParams(dimension_semantics=("arbitrary",),
                               vmem_limit_bytes=vmem1)
    cp2 = pltpu.CompilerParams(dimension_semantics=("arbitrary",),
                               vmem_limit_bytes=vmem2)

    h1, a_bf = pl.pallas_call(
        _layer1_cast_kernel,
        out_shape=(jax.ShapeDtypeStruct((n_pad, f_hid), jnp.bfloat16),
                   jax.ShapeDtypeStruct((n_rel, n_pad, n_pad), jnp.bfloat16)),
        grid=(n_tiles,),
        in_specs=[
            pl.BlockSpec((n_rel, tile_m, n_pad), lambda i: (0, i, 0)),
            pl.BlockSpec((n_pad, f_in), lambda i: (0, 0)),
            pl.BlockSpec((n_rel, f_in, f_hid), lambda i: (0, 0, 0)),
            pl.BlockSpec((tile_m, f_hid), lambda i: (i, 0)),
        ],
        out_specs=(pl.BlockSpec((tile_m, f_hid), lambda i: (i, 0)),
                   pl.BlockSpec((n_rel, tile_m, n_pad), lambda i: (0, i, 0))),
        compiler_params=cp1,
    )(a_p, x_p, w1_p, b1_p)

    out = pl.pallas_call(
        _layer2_kernel,
        out_shape=jax.ShapeDtypeStruct((g_pad, wc.shape[1]), jnp.float32),
        grid=(n_tiles,),
        in_specs=[
            pl.BlockSpec((n_rel, tile_m, n_pad), lambda i: (0, i, 0)),
            pl.BlockSpec((n_pad, f_hid), lambda i: (0, 0)),
            pl.BlockSpec((n_rel, f_hid, f_hid), lambda i: (0, 0, 0)),
            pl.BlockSpec((tile_m, f_hid), lambda i: (i, 0)),
            pl.BlockSpec((g_pad, tile_m), lambda i: (0, i)),
            pl.BlockSpec((f_hid, wc.shape[1]), lambda i: (0, 0)),
            pl.BlockSpec((8, wc.shape[1]), lambda i: (0, 0)),
        ],
        out_specs=pl.BlockSpec((g_pad, wc.shape[1]), lambda i: (0, 0)),
        scratch_shapes=[pltpu.VMEM((g_pad, f_hid), jnp.float32)],
        compiler_params=cp2,
    )(a_bf, h1, w2_p, b2_p, pool_p, wc_p, bc_p)

    return out[:n_graphs, :n_classes]


def kernel(a_norm, x, w1, b1_node, w2, b2_node, pool, wc, bc):
    return _forward(a_norm, x, w1, b1_node, w2, b2_node, pool, wc, bc,
                    tile_m=256)


# speedup vs baseline: 1.1312x; 1.1312x over previous
import math
from functools import partial

import jax
import jax.numpy as jnp
from jax.experimental import pallas as pl
from jax.experimental.pallas import tpu as pltpu


def _make_fused_kernel(t_hbm, k_cache, tile_m):
    def _fused_kernel(a_ref, x_ref, w1_ref, b1_ref, w2_ref, b2_ref, pool_ref,
                      wc_ref, bc_ref, out_ref, h1_s, hg_s, *maybe_abf):
        abf_s = maybe_abf[0] if k_cache else None
        p = pl.program_id(0)
        i = pl.program_id(1)
        n_rel = a_ref.shape[0]

        @pl.when(p == 0)
        def _layer1():
            a = a_ref[...]
            x = x_ref[...]
            acc = b1_ref[...]
            for r in range(n_rel):
                z = jnp.dot(a[r], x, preferred_element_type=jnp.float32)
                acc = acc + jnp.dot(z, w1_ref[r],
                                    preferred_element_type=jnp.float32)
            h1_s[pl.ds(i * tile_m, tile_m), :] = jnp.maximum(acc, 0.0)
            if k_cache:
                @pl.when(i >= t_hbm)
                def _park():
                    abf_s[:, pl.ds((i - t_hbm) * tile_m, tile_m), :] = (
                        a.astype(jnp.bfloat16))

        def _accum_part(part):
            @pl.when(i == 0)
            def _init():
                hg_s[...] = part

            @pl.when(i > 0)
            def _acc():
                hg_s[...] += part

        @pl.when(p == 1)
        def _layer2():
            @pl.when(i < t_hbm)
            def _from_hbm():
                h1 = h1_s[...]
                acc = b2_ref[...]
                for r in range(n_rel):
                    z = jnp.dot(a_ref[r], h1,
                                preferred_element_type=jnp.float32)
                    acc = acc + jnp.dot(z, w2_ref[r],
                                        preferred_element_type=jnp.float32)
                _accum_part(jnp.dot(pool_ref[...], acc,
                                    preferred_element_type=jnp.float32))

            if k_cache:
                @pl.when(i >= t_hbm)
                def _from_vmem():
                    h1 = h1_s[...]
                    acc = b2_ref[...]
                    for r in range(n_rel):
                        a_r = abf_s[r, pl.ds((i - t_hbm) * tile_m, tile_m), :]
                        z = jnp.dot(a_r.astype(jnp.float32), h1,
                                    preferred_element_type=jnp.float32)
                        acc = acc + jnp.dot(z, w2_ref[r],
                                            preferred_element_type=jnp.float32)
                    _accum_part(jnp.dot(pool_ref[...], acc,
                                        preferred_element_type=jnp.float32))

        @pl.when(jnp.logical_and(p == 1, i == pl.num_programs(1) - 1))
        def _classifier():
            out_ref[...] = (jnp.dot(hg_s[...], wc_ref[...],
                                    preferred_element_type=jnp.float32)
                            + bc_ref[0:1])

    return _fused_kernel


def _pad_to(a, shape):
    return jnp.pad(a, [(0, t - s) for s, t in zip(a.shape, shape)])


@partial(jax.jit, static_argnames=("tile_m",))
def _forward(a_norm, x, w1, b1_node, w2, b2_node, pool, wc, bc, *, tile_m=256):
    n_rel, n, _ = a_norm.shape
    f_in = x.shape[1]
    f_hid = w1.shape[2]
    n_graphs = pool.shape[0]
    n_classes = wc.shape[1]

    n_pad = tile_m * pl.cdiv(n, tile_m)
    g_pad = 8 * pl.cdiv(n_graphs, 8)
    n_tiles = n_pad // tile_m
    k_cache = 0 if n_tiles < 4 else min(2, n_tiles - 2)
    t_hbm = n_tiles - k_cache

    a_p = _pad_to(a_norm.astype(jnp.float32), (n_rel, n_pad, n_pad))
    x_p = _pad_to(x.astype(jnp.float32), (n_pad, f_in))
    w1_p = jnp.asarray(w1, jnp.float32)
    w2_p = jnp.asarray(w2, jnp.float32)
    b1_p = _pad_to(b1_node.astype(jnp.float32), (n_pad, f_hid))
    b2_p = _pad_to(b2_node.astype(jnp.float32), (n_pad, f_hid))
    pool_p = _pad_to(pool.astype(jnp.float32), (g_pad, n_pad))
    wc_p = jnp.asarray(wc, jnp.float32)
    bc_p = jnp.tile(jnp.asarray(bc, jnp.float32)[None, :], (8, 1))

    scratch_shapes = [
        pltpu.VMEM((n_pad, f_hid), jnp.float32),
        pltpu.VMEM((g_pad, f_hid), jnp.float32),
    ]
    if k_cache:
        scratch_shapes.append(
            pltpu.VMEM((n_rel, k_cache * tile_m, n_pad), jnp.bfloat16))

    slab_f32 = n_rel * tile_m * n_pad * 4
    scratch_bytes = (n_pad * f_hid * 4 + g_pad * f_hid * 4
                     + n_rel * k_cache * tile_m * n_pad * 2)
    resident = (n_pad * f_in * 4 + n_rel * f_hid * f_hid * 8
                + 4 * tile_m * f_hid * 4 + 4 * g_pad * (tile_m + f_hid) * 4)
    vmem_limit = int(min(max(2 * slab_f32 + scratch_bytes + resident
                             + (12 << 20), 32 << 20), 60000 << 10))
    cparams = pltpu.CompilerParams(
        dimension_semantics=("arbitrary", "arbitrary"),
        vmem_limit_bytes=vmem_limit)

    def _a_map(p, i):
        return (0, jnp.where(p == 1, jnp.minimum(i, t_hbm - 1), i), 0)

    out = pl.pallas_call(
        _make_fused_kernel(t_hbm, k_cache, tile_m),
        out_shape=jax.ShapeDtypeStruct((g_pad, wc.shape[1]), jnp.float32),
        grid=(2, n_tiles),
        in_specs=[
            pl.BlockSpec((n_rel, tile_m, n_pad), _a_map),
            pl.BlockSpec((n_pad, f_in), lambda p, i: (0, 0)),
            pl.BlockSpec((n_rel, f_in, f_hid), lambda p, i: (0, 0, 0)),
            pl.BlockSpec((tile_m, f_hid), lambda p, i: (i * (1 - p), 0)),
            pl.BlockSpec((n_rel, f_hid, f_hid), lambda p, i: (0, 0, 0)),
            pl.BlockSpec((tile_m, f_hid), lambda p, i: (i * p, 0)),
            pl.BlockSpec((g_pad, tile_m), lambda p, i: (0, i * p)),
            pl.BlockSpec((f_hid, wc.shape[1]), lambda p, i: (0, 0)),
            pl.BlockSpec((8, wc.shape[1]), lambda p, i: (0, 0)),
        ],
        out_specs=pl.BlockSpec((g_pad, wc.shape[1]), lambda p, i: (0, 0)),
        scratch_shapes=scratch_shapes,
        compiler_params=cparams,
    )(a_p, x_p, w1_p, b1_p, w2_p, b2_p, pool_p, wc_p, bc_p)

    return out[:n_graphs, :n_classes]


def kernel(a_norm, x, w1, b1_node, w2, b2_node, pool, wc, bc):
    return _forward(a_norm, x, w1, b1_node, w2, b2_node, pool, wc, bc,
                    tile_m=256)


# speedup vs baseline: 1.1711x; 1.0353x over previous
import math
from functools import partial

import jax
import jax.numpy as jnp
from jax.experimental import pallas as pl
from jax.experimental.pallas import tpu as pltpu


def _make_fused_kernel(t_hbm, k_cache, tile_m):
    def _fused_kernel(a_ref, x_ref, w1_ref, b1_ref, w2_ref, b2_ref, pool_ref,
                      wc_ref, bc_ref, out_ref, h1_s, hg_s, *maybe_abf):
        abf_s = maybe_abf[0] if k_cache else None
        p = pl.program_id(0)
        i = pl.program_id(1)
        n_rel = a_ref.shape[0]

        @pl.when(p == 0)
        def _layer1():
            x = x_ref[...]
            acc = b1_ref[...]
            for r in range(n_rel):
                z = jnp.dot(a_ref[r], x, preferred_element_type=jnp.float32)
                acc = acc + jnp.dot(z, w1_ref[r],
                                    preferred_element_type=jnp.float32)
            h1_s[pl.ds(i * tile_m, tile_m), :] = jnp.maximum(acc, 0.0)
            if k_cache:
                @pl.when(i >= t_hbm)
                def _park():
                    for r in range(n_rel):
                        abf_s[r, pl.ds((i - t_hbm) * tile_m, tile_m), :] = (
                            a_ref[r].astype(jnp.bfloat16))

        def _accum_part(part):
            @pl.when(i == 0)
            def _init():
                hg_s[...] = part

            @pl.when(i > 0)
            def _acc():
                hg_s[...] += part

        @pl.when(p == 1)
        def _layer2():
            @pl.when(i < t_hbm)
            def _from_hbm():
                h1 = h1_s[...]
                acc = b2_ref[...]
                for r in range(n_rel):
                    z = jnp.dot(a_ref[r], h1,
                                preferred_element_type=jnp.float32)
                    acc = acc + jnp.dot(z, w2_ref[r],
                                        preferred_element_type=jnp.float32)
                _accum_part(jnp.dot(pool_ref[...], acc,
                                    preferred_element_type=jnp.float32))

            if k_cache:
                @pl.when(i >= t_hbm)
                def _from_vmem():
                    h1 = h1_s[...]
                    acc = b2_ref[...]
                    for r in range(n_rel):
                        a_r = abf_s[r, pl.ds((i - t_hbm) * tile_m, tile_m), :]
                        z = jnp.dot(a_r.astype(jnp.float32), h1,
                                    preferred_element_type=jnp.float32)
                        acc = acc + jnp.dot(z, w2_ref[r],
                                            preferred_element_type=jnp.float32)
                    _accum_part(jnp.dot(pool_ref[...], acc,
                                        preferred_element_type=jnp.float32))

        @pl.when(jnp.logical_and(p == 1, i == pl.num_programs(1) - 1))
        def _classifier():
            out_ref[...] = (jnp.dot(hg_s[...], wc_ref[...],
                                    preferred_element_type=jnp.float32)
                            + bc_ref[0:1])

    return _fused_kernel


def _pad_to(a, shape):
    return jnp.pad(a, [(0, t - s) for s, t in zip(a.shape, shape)])


@partial(jax.jit, static_argnames=("tile_m",))
def _forward(a_norm, x, w1, b1_node, w2, b2_node, pool, wc, bc, *, tile_m=256):
    n_rel, n, _ = a_norm.shape
    f_in = x.shape[1]
    f_hid = w1.shape[2]
    n_graphs = pool.shape[0]
    n_classes = wc.shape[1]

    n_pad = tile_m * pl.cdiv(n, tile_m)
    g_pad = 8 * pl.cdiv(n_graphs, 8)
    n_tiles = n_pad // tile_m
    k_cache = 0 if n_tiles < 4 else min(3, n_tiles - 2)
    t_hbm = n_tiles - k_cache

    a_p = _pad_to(a_norm.astype(jnp.float32), (n_rel, n_pad, n_pad))
    x_p = _pad_to(x.astype(jnp.float32), (n_pad, f_in))
    w1_p = jnp.asarray(w1, jnp.float32)
    w2_p = jnp.asarray(w2, jnp.float32)
    b1_p = _pad_to(b1_node.astype(jnp.float32), (n_pad, f_hid))
    b2_p = _pad_to(b2_node.astype(jnp.float32), (n_pad, f_hid))
    pool_p = _pad_to(pool.astype(jnp.float32), (g_pad, n_pad))
    wc_p = jnp.asarray(wc, jnp.float32)
    bc_p = jnp.tile(jnp.asarray(bc, jnp.float32)[None, :], (8, 1))

    scratch_shapes = [
        pltpu.VMEM((n_pad, f_hid), jnp.float32),
        pltpu.VMEM((g_pad, f_hid), jnp.float32),
    ]
    if k_cache:
        scratch_shapes.append(
            pltpu.VMEM((n_rel, k_cache * tile_m, n_pad), jnp.bfloat16))

    slab_f32 = n_rel * tile_m * n_pad * 4
    scratch_bytes = (n_pad * f_hid * 4 + g_pad * f_hid * 4
                     + n_rel * k_cache * tile_m * n_pad * 2)
    resident = (n_pad * f_in * 4 + n_rel * f_hid * f_hid * 8
                + 4 * tile_m * f_hid * 4 + 4 * g_pad * (tile_m + f_hid) * 4)
    vmem_limit = int(min(max(2 * slab_f32 + scratch_bytes + resident
                             + (12 << 20), 32 << 20), 60000 << 10))
    cparams = pltpu.CompilerParams(
        dimension_semantics=("arbitrary", "arbitrary"),
        vmem_limit_bytes=vmem_limit)

    def _a_map(p, i):
        return (0, jnp.where(p == 1, jnp.minimum(i, t_hbm - 1), i), 0)

    out = pl.pallas_call(
        _make_fused_kernel(t_hbm, k_cache, tile_m),
        out_shape=jax.ShapeDtypeStruct((g_pad, wc.shape[1]), jnp.float32),
        grid=(2, n_tiles),
        in_specs=[
            pl.BlockSpec((n_rel, tile_m, n_pad), _a_map),
            pl.BlockSpec((n_pad, f_in), lambda p, i: (0, 0)),
            pl.BlockSpec((n_rel, f_in, f_hid), lambda p, i: (0, 0, 0)),
            pl.BlockSpec((tile_m, f_hid), lambda p, i: (i * (1 - p), 0)),
            pl.BlockSpec((n_rel, f_hid, f_hid), lambda p, i: (0, 0, 0)),
            pl.BlockSpec((tile_m, f_hid), lambda p, i: (i * p, 0)),
            pl.BlockSpec((g_pad, tile_m), lambda p, i: (0, i * p)),
            pl.BlockSpec((f_hid, wc.shape[1]), lambda p, i: (0, 0)),
            pl.BlockSpec((8, wc.shape[1]), lambda p, i: (0, 0)),
        ],
        out_specs=pl.BlockSpec((g_pad, wc.shape[1]), lambda p, i: (0, 0)),
        scratch_shapes=scratch_shapes,
        compiler_params=cparams,
    )(a_p, x_p, w1_p, b1_p, w2_p, b2_p, pool_p, wc_p, bc_p)

    return out[:n_graphs, :n_classes]


def kernel(a_norm, x, w1, b1_node, w2, b2_node, pool, wc, bc):
    return _forward(a_norm, x, w1, b1_node, w2, b2_node, pool, wc, bc,
                    tile_m=256)


# speedup vs baseline: 1.1780x; 1.0059x over previous
import math
from functools import partial

import jax
import jax.numpy as jnp
from jax.experimental import pallas as pl
from jax.experimental.pallas import tpu as pltpu


def _make_fused_kernel(t_hbm, k_cache, tile_m):
    def _fused_kernel(a_ref, x_ref, w1_ref, b1_ref, w2_ref, b2_ref, pool_ref,
                      wc_ref, bc_ref, out_ref, h1_s, hg_s, *maybe_abf):
        abf_s = maybe_abf[0] if k_cache else None
        p = pl.program_id(0)
        i = pl.program_id(1)
        n_rel = a_ref.shape[0]

        @pl.when(p == 0)
        def _layer1():
            x = x_ref[...]
            acc = b1_ref[...]
            for r in range(n_rel):
                z = jnp.dot(a_ref[r], x, preferred_element_type=jnp.float32)
                acc = acc + jnp.dot(z, w1_ref[r],
                                    preferred_element_type=jnp.float32)
            h1_s[pl.ds(i * tile_m, tile_m), :] = jnp.maximum(acc, 0.0)
            if k_cache:
                @pl.when(i >= t_hbm)
                def _park():
                    for r in range(n_rel):
                        abf_s[r, pl.ds((i - t_hbm) * tile_m, tile_m), :] = (
                            a_ref[r].astype(jnp.bfloat16))

        def _accum_part(part):
            @pl.when(i == 0)
            def _init():
                hg_s[...] = part

            @pl.when(i > 0)
            def _acc():
                hg_s[...] += part

        @pl.when(p == 1)
        def _layer2():
            @pl.when(i < t_hbm)
            def _from_hbm():
                h1 = h1_s[...]
                acc = b2_ref[...]
                for r in range(n_rel):
                    z = jnp.dot(a_ref[r], h1,
                                preferred_element_type=jnp.float32)
                    acc = acc + jnp.dot(z, w2_ref[r],
                                        preferred_element_type=jnp.float32)
                _accum_part(jnp.dot(pool_ref[...], acc,
                                    preferred_element_type=jnp.float32))

            if k_cache:
                @pl.when(i >= t_hbm)
                def _from_vmem():
                    h1 = h1_s[...]
                    acc = b2_ref[...]
                    for r in range(n_rel):
                        a_r = abf_s[r, pl.ds((i - t_hbm) * tile_m, tile_m), :]
                        z = jnp.dot(a_r.astype(jnp.float32), h1,
                                    preferred_element_type=jnp.float32)
                        acc = acc + jnp.dot(z, w2_ref[r],
                                            preferred_element_type=jnp.float32)
                    _accum_part(jnp.dot(pool_ref[...], acc,
                                        preferred_element_type=jnp.float32))

        @pl.when(jnp.logical_and(p == 1, i == pl.num_programs(1) - 1))
        def _classifier():
            out_ref[...] = (jnp.dot(hg_s[...], wc_ref[...],
                                    preferred_element_type=jnp.float32)
                            + bc_ref[0:1])

    return _fused_kernel


def _pad_to(a, shape):
    return jnp.pad(a, [(0, t - s) for s, t in zip(a.shape, shape)])


@partial(jax.jit, static_argnames=("tile_m",))
def _forward(a_norm, x, w1, b1_node, w2, b2_node, pool, wc, bc, *, tile_m=256):
    n_rel, n, _ = a_norm.shape
    f_in = x.shape[1]
    f_hid = w1.shape[2]
    n_graphs = pool.shape[0]
    n_classes = wc.shape[1]

    n_pad = tile_m * pl.cdiv(n, tile_m)
    g_pad = 8 * pl.cdiv(n_graphs, 8)
    n_tiles = n_pad // tile_m
    k_cache = 0 if n_tiles < 4 else min(4, n_tiles - 2)
    t_hbm = n_tiles - k_cache

    a_p = _pad_to(a_norm.astype(jnp.float32), (n_rel, n_pad, n_pad))
    x_p = _pad_to(x.astype(jnp.float32), (n_pad, f_in))
    w1_p = jnp.asarray(w1, jnp.float32)
    w2_p = jnp.asarray(w2, jnp.float32)
    b1_p = _pad_to(b1_node.astype(jnp.float32), (n_pad, f_hid))
    b2_p = _pad_to(b2_node.astype(jnp.float32), (n_pad, f_hid))
    pool_p = _pad_to(pool.astype(jnp.float32), (g_pad, n_pad))
    wc_p = jnp.asarray(wc, jnp.float32)
    bc_p = jnp.tile(jnp.asarray(bc, jnp.float32)[None, :], (8, 1))

    scratch_shapes = [
        pltpu.VMEM((n_pad, f_hid), jnp.float32),
        pltpu.VMEM((g_pad, f_hid), jnp.float32),
    ]
    if k_cache:
        scratch_shapes.append(
            pltpu.VMEM((n_rel, k_cache * tile_m, n_pad), jnp.bfloat16))

    slab_f32 = n_rel * tile_m * n_pad * 4
    scratch_bytes = (n_pad * f_hid * 4 + g_pad * f_hid * 4
                     + n_rel * k_cache * tile_m * n_pad * 2)
    resident = (n_pad * f_in * 4 + n_rel * f_hid * f_hid * 8
                + 4 * tile_m * f_hid * 4 + 4 * g_pad * (tile_m + f_hid) * 4)
    vmem_limit = int(min(max(2 * slab_f32 + scratch_bytes + resident
                             + (12 << 20), 32 << 20), 63 << 20))
    cparams = pltpu.CompilerParams(
        dimension_semantics=("arbitrary", "arbitrary"),
        vmem_limit_bytes=vmem_limit)

    def _a_map(p, i):
        return (0, jnp.where(p == 1, jnp.minimum(i, t_hbm - 1), i), 0)

    out = pl.pallas_call(
        _make_fused_kernel(t_hbm, k_cache, tile_m),
        out_shape=jax.ShapeDtypeStruct((g_pad, wc.shape[1]), jnp.float32),
        grid=(2, n_tiles),
        in_specs=[
            pl.BlockSpec((n_rel, tile_m, n_pad), _a_map),
            pl.BlockSpec((n_pad, f_in), lambda p, i: (0, 0)),
            pl.BlockSpec((n_rel, f_in, f_hid), lambda p, i: (0, 0, 0)),
            pl.BlockSpec((tile_m, f_hid), lambda p, i: (i * (1 - p), 0)),
            pl.BlockSpec((n_rel, f_hid, f_hid), lambda p, i: (0, 0, 0)),
            pl.BlockSpec((tile_m, f_hid), lambda p, i: (i * p, 0)),
            pl.BlockSpec((g_pad, tile_m), lambda p, i: (0, i * p)),
            pl.BlockSpec((f_hid, wc.shape[1]), lambda p, i: (0, 0)),
            pl.BlockSpec((8, wc.shape[1]), lambda p, i: (0, 0)),
        ],
        out_specs=pl.BlockSpec((g_pad, wc.shape[1]), lambda p, i: (0, 0)),
        scratch_shapes=scratch_shapes,
        compiler_params=cparams,
    )(a_p, x_p, w1_p, b1_p, w2_p, b2_p, pool_p, wc_p, bc_p)

    return out[:n_graphs, :n_classes]


def kernel(a_norm, x, w1, b1_node, w2, b2_node, pool, wc, bc):
    return _forward(a_norm, x, w1, b1_node, w2, b2_node, pool, wc, bc,
                    tile_m=256)


# speedup vs baseline: 1.1785x; 1.0005x over previous
import math
from functools import partial

import jax
import jax.numpy as jnp
from jax.experimental import pallas as pl
from jax.experimental.pallas import tpu as pltpu


def _make_fused_kernel(t_hbm, k_cache, tile_m):
    def _fused_kernel(a_ref, x_ref, w1_ref, b1_ref, w2_ref, b2_ref, pool_ref,
                      wc_ref, bc_ref, out_ref, h1_s, h1bf_s, hg_s, *maybe_abf):
        abf_s = maybe_abf[0] if k_cache else None
        p = pl.program_id(0)
        i = pl.program_id(1)
        n_rel = a_ref.shape[0]

        @pl.when(p == 0)
        def _layer1():
            x = x_ref[...]
            acc = b1_ref[...]
            for r in range(n_rel):
                z = jnp.dot(a_ref[r], x, preferred_element_type=jnp.float32)
                acc = acc + jnp.dot(z, w1_ref[r],
                                    preferred_element_type=jnp.float32)
            h1t = jnp.maximum(acc, 0.0)
            h1_s[pl.ds(i * tile_m, tile_m), :] = h1t
            h1bf_s[pl.ds(i * tile_m, tile_m), :] = h1t.astype(jnp.bfloat16)
            if k_cache:
                @pl.when(i >= t_hbm)
                def _park():
                    for r in range(n_rel):
                        abf_s[r, pl.ds((i - t_hbm) * tile_m, tile_m), :] = (
                            a_ref[r].astype(jnp.bfloat16))

        def _accum_part(part):
            @pl.when(i == 0)
            def _init():
                hg_s[...] = part

            @pl.when(i > 0)
            def _acc():
                hg_s[...] += part

        @pl.when(p == 1)
        def _layer2():
            @pl.when(i < t_hbm)
            def _from_hbm():
                h1 = h1_s[...]
                acc = b2_ref[...]
                for r in range(n_rel):
                    z = jnp.dot(a_ref[r], h1,
                                preferred_element_type=jnp.float32)
                    acc = acc + jnp.dot(z, w2_ref[r],
                                        preferred_element_type=jnp.float32)
                _accum_part(jnp.dot(pool_ref[...], acc,
                                    preferred_element_type=jnp.float32))

            if k_cache:
                @pl.when(i >= t_hbm)
                def _from_vmem():
                    h1b = h1bf_s[...]
                    acc = b2_ref[...]
                    for r in range(n_rel):
                        a_r = abf_s[r, pl.ds((i - t_hbm) * tile_m, tile_m), :]
                        z = jnp.dot(a_r, h1b,
                                    preferred_element_type=jnp.float32)
                        acc = acc + jnp.dot(z, w2_ref[r],
                                            preferred_element_type=jnp.float32)
                    _accum_part(jnp.dot(pool_ref[...], acc,
                                        preferred_element_type=jnp.float32))

        @pl.when(jnp.logical_and(p == 1, i == pl.num_programs(1) - 1))
        def _classifier():
            out_ref[...] = (jnp.dot(hg_s[...], wc_ref[...],
                                    preferred_element_type=jnp.float32)
                            + bc_ref[0:1])

    return _fused_kernel


def _pad_to(a, shape):
    return jnp.pad(a, [(0, t - s) for s, t in zip(a.shape, shape)])


@partial(jax.jit, static_argnames=("tile_m",))
def _forward(a_norm, x, w1, b1_node, w2, b2_node, pool, wc, bc, *, tile_m=256):
    n_rel, n, _ = a_norm.shape
    f_in = x.shape[1]
    f_hid = w1.shape[2]
    n_graphs = pool.shape[0]
    n_classes = wc.shape[1]

    n_pad = tile_m * pl.cdiv(n, tile_m)
    g_pad = 8 * pl.cdiv(n_graphs, 8)
    n_tiles = n_pad // tile_m
    k_cache = 0 if n_tiles < 4 else min(4, n_tiles - 2)
    t_hbm = n_tiles - k_cache

    a_p = _pad_to(a_norm.astype(jnp.float32), (n_rel, n_pad, n_pad))
    x_p = _pad_to(x.astype(jnp.float32), (n_pad, f_in))
    w1_p = jnp.asarray(w1, jnp.float32)
    w2_p = jnp.asarray(w2, jnp.float32)
    b1_p = _pad_to(b1_node.astype(jnp.float32), (n_pad, f_hid))
    b2_p = _pad_to(b2_node.astype(jnp.float32), (n_pad, f_hid))
    pool_p = _pad_to(pool.astype(jnp.float32), (g_pad, n_pad))
    wc_p = jnp.asarray(wc, jnp.float32)
    bc_p = jnp.tile(jnp.asarray(bc, jnp.float32)[None, :], (8, 1))

    scratch_shapes = [
        pltpu.VMEM((n_pad, f_hid), jnp.float32),
        pltpu.VMEM((n_pad, f_hid), jnp.bfloat16),
        pltpu.VMEM((g_pad, f_hid), jnp.float32),
    ]
    if k_cache:
        scratch_shapes.append(
            pltpu.VMEM((n_rel, k_cache * tile_m, n_pad), jnp.bfloat16))

    slab_f32 = n_rel * tile_m * n_pad * 4
    scratch_bytes = (n_pad * f_hid * 6 + g_pad * f_hid * 4
                     + n_rel * k_cache * tile_m * n_pad * 2)
    resident = (n_pad * f_in * 4 + n_rel * f_hid * f_hid * 8
                + 4 * tile_m * f_hid * 4 + 4 * g_pad * (tile_m + f_hid) * 4)
    vmem_limit = int(min(max(2 * slab_f32 + scratch_bytes + resident
                             + (12 << 20), 32 << 20), 63 << 20))
    cparams = pltpu.CompilerParams(
        dimension_semantics=("arbitrary", "arbitrary"),
        vmem_limit_bytes=vmem_limit)

    def _a_map(p, i):
        return (0, jnp.where(p == 1, jnp.minimum(i, t_hbm - 1), i), 0)

    out = pl.pallas_call(
        _make_fused_kernel(t_hbm, k_cache, tile_m),
        out_shape=jax.ShapeDtypeStruct((g_pad, wc.shape[1]), jnp.float32),
        grid=(2, n_tiles),
        in_specs=[
            pl.BlockSpec((n_rel, tile_m, n_pad), _a_map),
            pl.BlockSpec((n_pad, f_in), lambda p, i: (0, 0)),
            pl.BlockSpec((n_rel, f_in, f_hid), lambda p, i: (0, 0, 0)),
            pl.BlockSpec((tile_m, f_hid), lambda p, i: (i * (1 - p), 0)),
            pl.BlockSpec((n_rel, f_hid, f_hid), lambda p, i: (0, 0, 0)),
            pl.BlockSpec((tile_m, f_hid), lambda p, i: (i * p, 0)),
            pl.BlockSpec((g_pad, tile_m), lambda p, i: (0, i * p)),
            pl.BlockSpec((f_hid, wc.shape[1]), lambda p, i: (0, 0)),
            pl.BlockSpec((8, wc.shape[1]), lambda p, i: (0, 0)),
        ],
        out_specs=pl.BlockSpec((g_pad, wc.shape[1]), lambda p, i: (0, 0)),
        scratch_shapes=scratch_shapes,
        compiler_params=cparams,
    )(a_p, x_p, w1_p, b1_p, w2_p, b2_p, pool_p, wc_p, bc_p)

    return out[:n_graphs, :n_classes]


def kernel(a_norm, x, w1, b1_node, w2, b2_node, pool, wc, bc):
    return _forward(a_norm, x, w1, b1_node, w2, b2_node, pool, wc, bc,
                    tile_m=256)


# speedup vs baseline: 1.1787x; 1.0001x over previous
import math
from functools import partial

import jax
import jax.numpy as jnp
from jax.experimental import pallas as pl
from jax.experimental.pallas import tpu as pltpu


def _make_fused_kernel(t_hbm, k_cache, tile_m):
    def _fused_kernel(a_ref, x_ref, w1_ref, b1_ref, w2_ref, b2_ref, pool_ref,
                      wc_ref, bc_ref, out_ref, h1_s, hg_s, *maybe_abf):
        abf_s = maybe_abf[0] if k_cache else None
        p = pl.program_id(0)
        i = pl.program_id(1)
        n_rel = a_ref.shape[0]

        @pl.when(p == 0)
        def _layer1():
            x = x_ref[...]
            acc = b1_ref[...]
            for r in range(n_rel):
                z = jnp.dot(a_ref[r], x, preferred_element_type=jnp.float32)
                acc = acc + jnp.dot(z, w1_ref[r],
                                    preferred_element_type=jnp.float32)
            h1_s[pl.ds(i * tile_m, tile_m), :] = jnp.maximum(acc, 0.0)
            if k_cache:
                @pl.when(i >= t_hbm)
                def _park():
                    for r in range(n_rel):
                        abf_s[r, pl.ds((i - t_hbm) * tile_m, tile_m), :] = (
                            a_ref[r].astype(jnp.bfloat16))

        def _accum_part(part):
            @pl.when(i == 0)
            def _init():
                hg_s[...] = part

            @pl.when(i > 0)
            def _acc():
                hg_s[...] += part

        @pl.when(p == 1)
        def _layer2():
            @pl.when(i < t_hbm)
            def _from_hbm():
                h1 = h1_s[...]
                acc = b2_ref[...]
                for r in range(n_rel):
                    z = jnp.dot(a_ref[r], h1,
                                preferred_element_type=jnp.float32)
                    acc = acc + jnp.dot(z, w2_ref[r],
                                        preferred_element_type=jnp.float32)
                _accum_part(jnp.dot(pool_ref[...], acc,
                                    preferred_element_type=jnp.float32))

            if k_cache:
                @pl.when(i >= t_hbm)
                def _from_vmem():
                    h1 = h1_s[...]
                    acc = b2_ref[...]
                    for r in range(n_rel):
                        a_r = abf_s[r, pl.ds((i - t_hbm) * tile_m, tile_m), :]
                        z = jnp.dot(a_r.astype(jnp.float32), h1,
                                    preferred_element_type=jnp.float32)
                        acc = acc + jnp.dot(z, w2_ref[r],
                                            preferred_element_type=jnp.float32)
                    _accum_part(jnp.dot(pool_ref[...], acc,
                                        preferred_element_type=jnp.float32))

        @pl.when(jnp.logical_and(p == 1, i == pl.num_programs(1) - 1))
        def _classifier():
            out_ref[...] = (jnp.dot(hg_s[...], wc_ref[...],
                                    preferred_element_type=jnp.float32)
                            + bc_ref[0:1])

    return _fused_kernel


def _pad_to(a, shape):
    return jnp.pad(a, [(0, t - s) for s, t in zip(a.shape, shape)])


@partial(jax.jit, static_argnames=("tile_m",))
def _forward(a_norm, x, w1, b1_node, w2, b2_node, pool, wc, bc, *, tile_m=256):
    n_rel, n, _ = a_norm.shape
    f_in = x.shape[1]
    f_hid = w1.shape[2]
    n_graphs = pool.shape[0]
    n_classes = wc.shape[1]

    n_pad = tile_m * pl.cdiv(n, tile_m)
    g_pad = 8 * pl.cdiv(n_graphs, 8)
    n_tiles = n_pad // tile_m
    k_cache = 0 if n_tiles < 4 else min(4, n_tiles - 2)
    t_hbm = n_tiles - k_cache

    a_p = _pad_to(a_norm.astype(jnp.float32), (n_rel, n_pad, n_pad))
    x_p = _pad_to(x.astype(jnp.float32), (n_pad, f_in))
    w1_p = jnp.asarray(w1, jnp.float32)
    w2_p = jnp.asarray(w2, jnp.float32)
    b1_p = _pad_to(b1_node.astype(jnp.float32), (n_pad, f_hid))
    b2_p = _pad_to(b2_node.astype(jnp.float32), (n_pad, f_hid))
    pool_p = _pad_to(pool.astype(jnp.float32), (g_pad, n_pad))
    wc_p = jnp.asarray(wc, jnp.float32)
    bc_p = jnp.tile(jnp.asarray(bc, jnp.float32)[None, :], (8, 1))

    scratch_shapes = [
        pltpu.VMEM((n_pad, f_hid), jnp.float32),
        pltpu.VMEM((g_pad, f_hid), jnp.float32),
    ]
    if k_cache:
        scratch_shapes.append(
            pltpu.VMEM((n_rel, k_cache * tile_m, n_pad), jnp.bfloat16))

    slab_f32 = n_rel * tile_m * n_pad * 4
    scratch_bytes = (n_pad * f_hid * 4 + g_pad * f_hid * 4
                     + n_rel * k_cache * tile_m * n_pad * 2)
    resident = (n_pad * f_in * 4 + n_rel * f_hid * f_hid * 8
                + 4 * tile_m * f_hid * 4 + 4 * g_pad * (tile_m + f_hid) * 4)
    vmem_limit = int(min(max(2 * slab_f32 + scratch_bytes + resident
                             + (12 << 20), 32 << 20), 63 << 20))
    cparams = pltpu.CompilerParams(
        dimension_semantics=("arbitrary", "arbitrary"),
        vmem_limit_bytes=vmem_limit)

    def _a_map(p, i):
        return (0, jnp.where(p == 1, jnp.minimum(i, t_hbm - 1), i), 0)

    out = pl.pallas_call(
        _make_fused_kernel(t_hbm, k_cache, tile_m),
        out_shape=jax.ShapeDtypeStruct((g_pad, wc.shape[1]), jnp.float32),
        grid=(2, n_tiles),
        in_specs=[
            pl.BlockSpec((n_rel, tile_m, n_pad), _a_map),
            pl.BlockSpec((n_pad, f_in), lambda p, i: (0, 0)),
            pl.BlockSpec((n_rel, f_in, f_hid), lambda p, i: (0, 0, 0)),
            pl.BlockSpec((tile_m, f_hid), lambda p, i: (i * (1 - p), 0)),
            pl.BlockSpec((n_rel, f_hid, f_hid), lambda p, i: (0, 0, 0)),
            pl.BlockSpec((tile_m, f_hid), lambda p, i: (i * p, 0)),
            pl.BlockSpec((g_pad, tile_m), lambda p, i: (0, i * p)),
            pl.BlockSpec((f_hid, wc.shape[1]), lambda p, i: (0, 0)),
            pl.BlockSpec((8, wc.shape[1]), lambda p, i: (0, 0)),
        ],
        out_specs=pl.BlockSpec((g_pad, wc.shape[1]), lambda p, i: (0, 0)),
        scratch_shapes=scratch_shapes,
        compiler_params=cparams,
    )(a_p, x_p, w1_p, b1_p, w2_p, b2_p, pool_p, wc_p, bc_p)

    return out[:n_graphs, :n_classes]


def kernel(a_norm, x, w1, b1_node, w2, b2_node, pool, wc, bc):
    return _forward(a_norm, x, w1, b1_node, w2, b2_node, pool, wc, bc,
                    tile_m=256)


# speedup vs baseline: 1.1822x; 1.0030x over previous
from functools import partial

import jax
import jax.numpy as jnp
from jax.experimental import pallas as pl
from jax.experimental.pallas import tpu as pltpu


def _make_fused_kernel(t_hbm, k_cache, tile_m):
    def _fused_kernel(a_ref, x_ref, w1_ref, b1_ref, w2_ref, b2_ref, pool_ref,
                      wc_ref, bc_ref, out_ref, h1_s, hg_s, *maybe_abf):
        abf_s = maybe_abf[0] if k_cache else None
        p = pl.program_id(0)
        i = pl.program_id(1)
        n_rel = a_ref.shape[0]

        @pl.when(p == 0)
        def _layer1():
            x = x_ref[...]
            acc = b1_ref[...]
            for r in range(n_rel):
                z = jnp.dot(a_ref[r], x, preferred_element_type=jnp.float32)
                acc = acc + jnp.dot(z, w1_ref[r],
                                    preferred_element_type=jnp.float32)
            h1_s[pl.ds(i * tile_m, tile_m), :] = jnp.maximum(acc, 0.0)
            if k_cache:
                @pl.when(i >= t_hbm)
                def _park():
                    for r in range(n_rel):
                        abf_s[r, pl.ds((i - t_hbm) * tile_m, tile_m), :] = (
                            a_ref[r].astype(jnp.bfloat16))

        def _accum_part(part):
            @pl.when(i == 0)
            def _init():
                hg_s[...] = part

            @pl.when(i > 0)
            def _acc():
                hg_s[...] += part

        @pl.when(p == 1)
        def _layer2():
            @pl.when(i < t_hbm)
            def _from_hbm():
                h1 = h1_s[...]
                acc = b2_ref[...]
                for r in range(n_rel):
                    z = jnp.dot(a_ref[r], h1,
                                preferred_element_type=jnp.float32)
                    acc = acc + jnp.dot(z, w2_ref[r],
                                        preferred_element_type=jnp.float32)
                _accum_part(jnp.dot(pool_ref[...], acc,
                                    preferred_element_type=jnp.float32))

            if k_cache:
                @pl.when(i >= t_hbm)
                def _from_vmem():
                    h1 = h1_s[...]
                    acc = b2_ref[...]
                    for r in range(n_rel):
                        a_r = abf_s[r, pl.ds((i - t_hbm) * tile_m, tile_m), :]
                        z = jnp.dot(a_r.astype(jnp.float32), h1,
                                    preferred_element_type=jnp.float32)
                        acc = acc + jnp.dot(z, w2_ref[r],
                                            preferred_element_type=jnp.float32)
                    _accum_part(jnp.dot(pool_ref[...], acc,
                                        preferred_element_type=jnp.float32))

        @pl.when(jnp.logical_and(p == 1, i == pl.num_programs(1) - 1))
        def _classifier():
            out_ref[...] = (jnp.dot(hg_s[...], wc_ref[...],
                                    preferred_element_type=jnp.float32)
                            + bc_ref[0:1])

    return _fused_kernel


def _pad_to(a, shape):
    return jnp.pad(a, [(0, t - s) for s, t in zip(a.shape, shape)])


@partial(jax.jit, static_argnames=("tile_m",))
def _forward(a_norm, x, w1, b1_node, w2, b2_node, pool, wc, bc, *, tile_m=256):
    n_rel, n, _ = a_norm.shape
    f_in = x.shape[1]
    f_hid = w1.shape[2]
    n_graphs = pool.shape[0]
    n_classes = wc.shape[1]

    n_pad = tile_m * pl.cdiv(n, tile_m)
    g_pad = 8 * pl.cdiv(n_graphs, 8)
    n_tiles = n_pad // tile_m
    k_cache = 0 if n_tiles < 4 else min(4, n_tiles - 2)
    t_hbm = n_tiles - k_cache

    a_p = _pad_to(a_norm.astype(jnp.float32), (n_rel, n_pad, n_pad))
    x_p = _pad_to(x.astype(jnp.float32), (n_pad, f_in))
    w1_p = jnp.asarray(w1, jnp.float32)
    w2_p = jnp.asarray(w2, jnp.float32)
    b1_p = _pad_to(b1_node.astype(jnp.float32), (n_pad, f_hid))
    b2_p = _pad_to(b2_node.astype(jnp.float32), (n_pad, f_hid))
    pool_p = _pad_to(pool.astype(jnp.float32), (g_pad, n_pad))
    wc_p = jnp.asarray(wc, jnp.float32)
    bc_p = jnp.tile(jnp.asarray(bc, jnp.float32)[None, :], (8, 1))

    scratch_shapes = [
        pltpu.VMEM((n_pad, f_hid), jnp.float32),
        pltpu.VMEM((g_pad, f_hid), jnp.float32),
    ]
    if k_cache:
        scratch_shapes.append(
            pltpu.VMEM((n_rel, k_cache * tile_m, n_pad), jnp.bfloat16))

    slab_f32 = n_rel * tile_m * n_pad * 4
    scratch_bytes = (n_pad * f_hid * 4 + g_pad * f_hid * 4
                     + n_rel * k_cache * tile_m * n_pad * 2)
    resident = (n_pad * f_in * 4 + n_rel * f_hid * f_hid * 8
                + 4 * tile_m * f_hid * 4 + 4 * g_pad * (tile_m + f_hid) * 4)
    vmem_limit = int(min(max(2 * slab_f32 + scratch_bytes + resident
                             + (12 << 20), 32 << 20), 63 << 20))
    cparams = pltpu.CompilerParams(
        dimension_semantics=("arbitrary", "arbitrary"),
        vmem_limit_bytes=vmem_limit)

    def _a_map(p, i):
        return (0, jnp.where(p == 1, jnp.minimum(i, t_hbm - 1), i), 0)

    out = pl.pallas_call(
        _make_fused_kernel(t_hbm, k_cache, tile_m),
        out_shape=jax.ShapeDtypeStruct((g_pad, wc.shape[1]), jnp.float32),
        grid=(2, n_tiles),
        in_specs=[
            pl.BlockSpec((n_rel, tile_m, n_pad), _a_map),
            pl.BlockSpec((n_pad, f_in), lambda p, i: (0, 0)),
            pl.BlockSpec((n_rel, f_in, f_hid), lambda p, i: (0, 0, 0)),
            pl.BlockSpec((tile_m, f_hid), lambda p, i: (i * (1 - p), 0)),
            pl.BlockSpec((n_rel, f_hid, f_hid), lambda p, i: (0, 0, 0)),
            pl.BlockSpec((tile_m, f_hid), lambda p, i: (i * p, 0)),
            pl.BlockSpec((g_pad, tile_m), lambda p, i: (0, i * p)),
            pl.BlockSpec((f_hid, wc.shape[1]), lambda p, i: (0, 0)),
            pl.BlockSpec((8, wc.shape[1]), lambda p, i: (0, 0)),
        ],
        out_specs=pl.BlockSpec((g_pad, wc.shape[1]), lambda p, i: (0, 0)),
        scratch_shapes=scratch_shapes,
        compiler_params=cparams,
    )(a_p, x_p, w1_p, b1_p, w2_p, b2_p, pool_p, wc_p, bc_p)

    return out[:n_graphs, :n_classes]


def kernel(a_norm, x, w1, b1_node, w2, b2_node, pool, wc, bc):
    return _forward(a_norm, x, w1, b1_node, w2, b2_node, pool, wc, bc,
                    tile_m=256)
